# baseline (speedup 1.0000x reference)
import functools

import jax
import jax.numpy as jnp
from jax import lax
from jax.experimental import pallas as pl
from jax.experimental.pallas import tpu as pltpu
from jax.experimental.pallas import tpu_sc as plsc

S, B, D = 4096, 4, 1024
L = 16
NC, NS = 2, 16
NW = NC * NS
ROWS_W = S // NW
CH = 4
NBUF = 4
P = 2
W = NBUF - P
G = ROWS_W // CH

_mesh = plsc.VectorSubcoreMesh(core_axis_name="c", subcore_axis_name="s")


def _sc_body(x_hbm, pe_hbm, o_hbm, buf, pebuf, *sems):
    in_sems = sems[0:NBUF]
    pe_sems = sems[NBUF:2 * NBUF]
    out_sems = sems[2 * NBUF:3 * NBUF]
    wid = lax.axis_index("s") * NC + lax.axis_index("c")
    base = wid * ROWS_W

    def issue_in(j, slot):
        s0 = base + j * CH
        pltpu.async_copy(x_hbm.at[pl.ds(s0, CH)], buf.at[slot], in_sems[slot])
        pltpu.async_copy(pe_hbm.at[pl.ds(s0, CH)], pebuf.at[slot], pe_sems[slot])

    def wait_in(j, slot):
        s0 = base + j * CH
        pltpu.make_async_copy(x_hbm.at[pl.ds(s0, CH)], buf.at[slot], in_sems[slot]).wait()
        pltpu.make_async_copy(pe_hbm.at[pl.ds(s0, CH)], pebuf.at[slot], pe_sems[slot]).wait()

    def issue_out(j, slot):
        s0 = base + j * CH
        pltpu.async_copy(buf.at[slot], o_hbm.at[pl.ds(s0, CH)], out_sems[slot])

    def wait_out(j, slot):
        s0 = base + j * CH
        pltpu.make_async_copy(buf.at[slot], o_hbm.at[pl.ds(s0, CH)], out_sems[slot]).wait()

    UNROLL = 4

    def compute(slot):
        def dv_body(dv, carry):
            for u in range(UNROLL):
                off = pl.multiple_of((dv * UNROLL + u) * L, L)
                for s in range(CH):
                    pe_vec = pebuf[slot, s, pl.ds(off, L)]
                    for b in range(B):
                        plsc.addupdate(buf.at[slot, s, b, pl.ds(off, L)], pe_vec)
            return carry

        lax.fori_loop(0, D // (L * UNROLL), dv_body, 0)

    def step(j, slot, do_wait_out, do_issue_in):
        pslot = (slot + P) % NBUF
        if do_wait_out:
            wait_out(j - W, pslot)
        if do_issue_in:
            issue_in(j + P, pslot)
        wait_in(j, slot)
        compute(slot)
        issue_out(j, slot)

    for j in range(P):
        issue_in(j, j)

    for j in range(W):
        step(j, j % NBUF, do_wait_out=False, do_issue_in=True)

    n_main = (G - P - W) // NBUF
    def main_body(g0, carry):
        for k in range(NBUF):
            step(g0 + k, (W + k) % NBUF, do_wait_out=True, do_issue_in=True)
        return carry

    lax.fori_loop(0, n_main, lambda m, c: main_body(W + m * NBUF, c), 0)

    for j in range(W + n_main * NBUF, G):
        step(j, j % NBUF, do_wait_out=True, do_issue_in=(j + P <= G - 1))

    for j in range(G - W, G):
        wait_out(j, j % NBUF)


def kernel(input, pe_table):
    k = functools.partial(
        pl.kernel,
        mesh=_mesh,
        out_type=jax.ShapeDtypeStruct((S, B, D), jnp.float32),
        scratch_types=[
            pltpu.VMEM((NBUF, CH, B, D), jnp.float32),
            pltpu.VMEM((NBUF, CH, D), jnp.float32),
        ] + [pltpu.SemaphoreType.DMA] * (3 * NBUF),
    )(_sc_body)
    return k(input, pe_table)


# speedup vs baseline: 1.0621x; 1.0621x over previous
import functools

import jax
import jax.numpy as jnp
from jax import lax
from jax.experimental import pallas as pl
from jax.experimental.pallas import tpu as pltpu
from jax.experimental.pallas import tpu_sc as plsc

S, B, D = 4096, 4, 1024
L = 16
NC, NS = 2, 16
NW = NC * NS
ROWS_W = S // NW
CH = 4
NBUF = 4
P = 2
W = NBUF - P
G = ROWS_W // CH

_mesh = plsc.VectorSubcoreMesh(core_axis_name="c", subcore_axis_name="s")


def _sc_body(x_hbm, pe_hbm, o_hbm, buf, pebuf, *sems):
    in_sems = sems[0:NBUF]
    pe_sems = sems[NBUF:2 * NBUF]
    out_sems = sems[2 * NBUF:3 * NBUF]
    wid = lax.axis_index("s") * NC + lax.axis_index("c")
    base = wid * ROWS_W

    def issue_in(j, slot):
        s0 = base + j * CH
        pltpu.async_copy(x_hbm.at[pl.ds(s0, CH)], buf.at[slot], in_sems[slot])
        pltpu.async_copy(pe_hbm.at[pl.ds(s0, CH)], pebuf.at[slot], pe_sems[slot])

    def wait_in(j, slot):
        s0 = base + j * CH
        pltpu.make_async_copy(x_hbm.at[pl.ds(s0, CH)], buf.at[slot], in_sems[slot]).wait()
        pltpu.make_async_copy(pe_hbm.at[pl.ds(s0, CH)], pebuf.at[slot], pe_sems[slot]).wait()

    def issue_out(j, slot):
        s0 = base + j * CH
        pltpu.async_copy(buf.at[slot], o_hbm.at[pl.ds(s0, CH)], out_sems[slot])

    def wait_out(j, slot):
        s0 = base + j * CH
        pltpu.make_async_copy(buf.at[slot], o_hbm.at[pl.ds(s0, CH)], out_sems[slot]).wait()

    UNROLL = 1

    def compute(slot):
        def dv_body(dv, carry):
            for u in range(UNROLL):
                off = pl.multiple_of((dv * UNROLL + u) * L, L)
                for s in range(CH):
                    pe_vec = pebuf[slot, s, pl.ds(off, L)]
                    for b in range(B):
                        plsc.addupdate(buf.at[slot, s, b, pl.ds(off, L)], pe_vec)
            return carry

        lax.fori_loop(0, D // (L * UNROLL), dv_body, 0)

    def step(j, slot, do_wait_out, do_issue_in):
        pslot = (slot + P) % NBUF
        if do_wait_out:
            wait_out(j - W, pslot)
        if do_issue_in:
            issue_in(j + P, pslot)
        wait_in(j, slot)
        compute(slot)
        issue_out(j, slot)

    for j in range(P):
        issue_in(j, j)

    for j in range(W):
        step(j, j % NBUF, do_wait_out=False, do_issue_in=True)

    n_main = (G - P - W) // NBUF
    def main_body(g0, carry):
        for k in range(NBUF):
            step(g0 + k, (W + k) % NBUF, do_wait_out=True, do_issue_in=True)
        return carry

    lax.fori_loop(0, n_main, lambda m, c: main_body(W + m * NBUF, c), 0)

    for j in range(W + n_main * NBUF, G):
        step(j, j % NBUF, do_wait_out=True, do_issue_in=(j + P <= G - 1))

    for j in range(G - W, G):
        wait_out(j, j % NBUF)


def kernel(input, pe_table):
    k = functools.partial(
        pl.kernel,
        mesh=_mesh,
        out_type=jax.ShapeDtypeStruct((S, B, D), jnp.float32),
        scratch_types=[
            pltpu.VMEM((NBUF, CH, B, D), jnp.float32),
            pltpu.VMEM((NBUF, CH, D), jnp.float32),
        ] + [pltpu.SemaphoreType.DMA] * (3 * NBUF),
    )(_sc_body)
    return k(input, pe_table)
